# Initial kernel scaffold; baseline (speedup 1.0000x reference)
import jax
import jax.numpy as jnp
from jax.experimental import pallas as pl


def kernel(x, edge_index, batch, W1, b1, g1, be1, W2, b2, g2, be2, W3, b3, g3, be3, Wm1, bm1, Wm2, bm2):
    raise NotImplementedError("write your pallas kernel here")


# speedup vs baseline: 15.1154x; 15.1154x over previous
import functools

import jax
import jax.numpy as jnp
from jax import lax
from jax.experimental import pallas as pl
from jax.experimental.pallas import tpu as pltpu
from jax.experimental.pallas import tpu_sc as plsc

_G = 64
_K = 80
_F = 128


def _sc_mesh():
    return plsc.VectorSubcoreMesh(core_axis_name="c", subcore_axis_name="s")


@functools.lru_cache(maxsize=None)
def _degree_kernel(NPAD, E):
    NW = 32
    NCH = E // NW // _K
    DF = _F
    RPT = NPAD // 16

    @functools.partial(
        pl.kernel,
        mesh=_sc_mesh(),
        out_type=jax.ShapeDtypeStruct((2, 16, RPT, DF), jnp.float32),
        scratch_types=[
            pltpu.VMEM((NCH, _K), jnp.int32),
            pltpu.VMEM((_K, DF), jnp.float32),
            pltpu.VMEM_SHARED((NPAD, DF), jnp.float32),
            pltpu.SemaphoreType.DMA,
        ],
    )
    def kern(dst3, ones, zeros, out, dst_v, ones_v, deg_sh, sem):
        c = lax.axis_index("c")
        s = lax.axis_index("s")
        wid = s * 2 + c
        pltpu.sync_copy(dst3.at[wid], dst_v)
        pltpu.sync_copy(ones, ones_v)
        r0 = s * RPT
        pltpu.sync_copy(zeros, deg_sh.at[pl.ds(r0, RPT)])
        plsc.subcore_barrier()

        def body(i, t):
            pltpu.sync_copy(ones_v, deg_sh.at[dst_v.at[i]], add=True)
            return t

        lax.fori_loop(0, NCH, body, 0)
        plsc.subcore_barrier()
        pltpu.sync_copy(deg_sh.at[pl.ds(r0, RPT)], out.at[c, s])

    return kern


@functools.lru_cache(maxsize=None)
def _edge_agg_kernel(NPAD, E):
    NW = 32
    NCH = E // NW // _K
    RPT = NPAD // 16

    @functools.partial(
        pl.kernel,
        mesh=_sc_mesh(),
        out_type=jax.ShapeDtypeStruct((2, 16, RPT, _F), jnp.float32),
        scratch_types=[
            pltpu.VMEM((NCH, _K), jnp.int32),
            pltpu.VMEM((NCH, _K), jnp.int32),
            pltpu.VMEM((_K, _F), jnp.float32),
            pltpu.VMEM_SHARED((NPAD, _F), jnp.float32),
            pltpu.SemaphoreType.DMA,
        ],
    )
    def kern(hs, src3, dst3, zeros, out, src_v, dst_v, rows_v, agg_sh, sem):
        c = lax.axis_index("c")
        s = lax.axis_index("s")
        wid = s * 2 + c
        pltpu.sync_copy(src3.at[wid], src_v)
        pltpu.sync_copy(dst3.at[wid], dst_v)
        r0 = s * RPT
        pltpu.sync_copy(zeros, agg_sh.at[pl.ds(r0, RPT)])
        plsc.subcore_barrier()

        def body(i, t):
            pltpu.async_copy(hs.at[src_v.at[i]], rows_v, sem).wait()
            pltpu.sync_copy(rows_v, agg_sh.at[dst_v.at[i]], add=True)
            return t

        lax.fori_loop(0, NCH, body, 0)
        plsc.subcore_barrier()
        pltpu.sync_copy(agg_sh.at[pl.ds(r0, RPT)], out.at[c, s])

    return kern


def _tc1_body(degp_ref, x_ref, w_ref, hs_ref, dis_ref):
    deg = degp_ref[0][:, 0:1] + degp_ref[1][:, 0:1] + 1.0
    dis = lax.rsqrt(deg)
    dis_ref[...] = dis
    hs_ref[...] = jnp.dot(
        x_ref[...], w_ref[...], preferred_element_type=jnp.float32) * dis


def _bn_relu(h_in, aggp_ref, hs_ref, dis_ref, b_ref, g_ref, be_ref):
    tot = (aggp_ref[0][:, :h_in] + aggp_ref[1][:, :h_in]
           + hs_ref[:, :h_in]) * dis_ref[...] + b_ref[...]
    m = jnp.mean(tot, axis=0, keepdims=True)
    d = tot - m
    v = jnp.mean(d * d, axis=0, keepdims=True)
    return jnp.maximum(d * lax.rsqrt(v + 1e-5) * g_ref[...] + be_ref[...], 0.0)


def _tc_mid_body(h_in, aggp_ref, hs_ref, dis_ref, b_ref, g_ref, be_ref,
                 w_ref, out_ref):
    act = _bn_relu(h_in, aggp_ref, hs_ref, dis_ref, b_ref, g_ref, be_ref)
    hn = jnp.dot(act, w_ref[...], preferred_element_type=jnp.float32) * dis_ref[...]
    n, h_out = hn.shape
    out_ref[...] = jnp.concatenate(
        [hn, jnp.zeros((n, _F - h_out), jnp.float32)], axis=1)


def _tc_post_body(h_in, aggp_ref, hs_ref, dis_ref, b_ref, g_ref, be_ref,
                  batch_ref, wm1_ref, bm1_ref, wm2_ref, bm2_ref, out_ref):
    act = _bn_relu(h_in, aggp_ref, hs_ref, dis_ref, b_ref, g_ref, be_ref)
    n = act.shape[0]
    gids = lax.broadcasted_iota(jnp.int32, (_G, n), 0)
    onehot_t = (gids == batch_ref[...]).astype(jnp.float32)
    ssum = jnp.dot(onehot_t, act, preferred_element_type=jnp.float32)
    cnt = jnp.dot(onehot_t, jnp.ones((n, 1), jnp.float32),
                  preferred_element_type=jnp.float32)
    pooled = ssum / jnp.maximum(cnt, 1.0)
    z = jnp.maximum(
        jnp.dot(pooled, wm1_ref[...], preferred_element_type=jnp.float32)
        + bm1_ref[...], 0.0)
    out_ref[...] = (
        jnp.dot(z, wm2_ref[...], preferred_element_type=jnp.float32)
        + bm2_ref[...])


def kernel(x, edge_index, batch, W1, b1, g1, be1, W2, b2, g2, be2,
           W3, b3, g3, be3, Wm1, bm1, Wm2, bm2):
    N, D = x.shape
    E = edge_index.shape[1]
    H1 = W1.shape[1]
    H2 = W2.shape[1]
    C = Wm2.shape[1]

    NW = 32
    NCH = E // NW // _K
    NPAD = ((N + 127) // 128) * 128
    RPT = NPAD // 16

    src3 = edge_index[0].reshape(NW, NCH, _K)
    dst3 = edge_index[1].reshape(NW, NCH, _K)
    batch_row = batch.reshape(1, N)

    ones_deg = jnp.ones((_K, _F), jnp.float32)
    zeros_f = jnp.zeros((RPT, _F), jnp.float32)

    degp = (_degree_kernel(NPAD, E)(dst3, ones_deg, zeros_f)
            .reshape(2, NPAD, _F)[:, :N, :8])

    hs1, dis = pl.pallas_call(
        _tc1_body,
        out_shape=(
            jax.ShapeDtypeStruct((N, H1), jnp.float32),
            jax.ShapeDtypeStruct((N, 1), jnp.float32),
        ),
    )(degp, x, W1)

    agg = _edge_agg_kernel(NPAD, E)
    agg1 = agg(hs1, src3, dst3, zeros_f).reshape(2, NPAD, _F)[:, :N]

    def tc_mid(h_in, aggp, hs, b, g, be, W):
        return pl.pallas_call(
            functools.partial(_tc_mid_body, h_in),
            out_shape=jax.ShapeDtypeStruct((N, _F), jnp.float32),
        )(aggp, hs, dis, b.reshape(1, -1), g.reshape(1, -1),
          be.reshape(1, -1), W)

    hs2 = tc_mid(H1, agg1, hs1, b1, g1, be1, W2)
    agg2 = agg(hs2, src3, dst3, zeros_f).reshape(2, NPAD, _F)[:, :N]
    hs3 = tc_mid(H2, agg2, hs2, b2, g2, be2, W3)
    agg3 = agg(hs3, src3, dst3, zeros_f).reshape(2, NPAD, _F)[:, :N]

    out = pl.pallas_call(
        functools.partial(_tc_post_body, H2),
        out_shape=jax.ShapeDtypeStruct((_G, C), jnp.float32),
    )(agg3, hs3, dis, b3.reshape(1, -1), g3.reshape(1, -1),
      be3.reshape(1, -1), batch_row, Wm1, bm1.reshape(1, -1),
      Wm2, bm2.reshape(1, -1))

    return out
